# baseline (speedup 1.0000x reference)
import functools

import jax
import jax.numpy as jnp
from jax import lax
from jax.experimental import pallas as pl
from jax.experimental.pallas import tpu as pltpu
from jax.experimental.pallas import tpu_sc as plsc

N_NODES = 10000
N_HIDDEN = 64
N_ETYPES = 4

NC = 2
NS = 16
NW = NC * NS
CHUNK = 128
ACC_ROWS = 10112
RPS = ACC_ROWS // NS
DUMMY_DST = N_NODES


_NBUF = 2


@functools.cache
def _make_sc_agg(cpw: int, table_rows: int):
    assert cpw % _NBUF == 0
    mesh = plsc.VectorSubcoreMesh(core_axis_name="c", subcore_axis_name="s")

    @functools.partial(
        pl.kernel,
        mesh=mesh,
        compiler_params=pltpu.CompilerParams(use_tc_tiling_on_sc=False),
        out_type=jax.ShapeDtypeStruct((NC, ACC_ROWS, N_HIDDEN), jnp.float32),
        scratch_types=[
            pltpu.VMEM((cpw, CHUNK), jnp.int32),
            pltpu.VMEM((cpw, CHUNK), jnp.int32),
            [pltpu.VMEM((CHUNK, N_HIDDEN), jnp.float32)] * _NBUF,
            pltpu.VMEM((RPS, N_HIDDEN), jnp.float32),
            pltpu.VMEM_SHARED((ACC_ROWS, N_HIDDEN), jnp.float32),
            [pltpu.SemaphoreType.DMA] * _NBUF,
            [pltpu.SemaphoreType.DMA] * _NBUF,
        ],
    )
    def sc_agg(rows_hbm, dst_hbm, table_hbm, out_hbm,
               rowbuf, dstbuf, gbufs, zbuf, acc, gsems, ssems):
        c = lax.axis_index("c")
        s = lax.axis_index("s")
        w = c * NS + s

        cp_r = pltpu.async_copy(rows_hbm.at[w], rowbuf, gsems[0])
        cp_d = pltpu.async_copy(dst_hbm.at[w], dstbuf, gsems[1])

        def _zrow(i, carry):
            for c4 in range(N_HIDDEN // 16):
                zbuf[i, pl.ds(c4 * 16, 16)] = jnp.zeros((16,), jnp.float32)
            return carry
        lax.fori_loop(0, RPS, _zrow, 0)
        pltpu.sync_copy(zbuf, acc.at[pl.ds(s * RPS, RPS)])
        cp_r.wait()
        cp_d.wait()
        plsc.subcore_barrier()

        for b in range(_NBUF):
            pltpu.async_copy(table_hbm.at[rowbuf.at[b]], gbufs[b], gsems[b])

        def _step(t, carry):
            j0 = t * _NBUF
            for b in range(_NBUF):
                j = j0 + b
                pltpu.make_async_copy(table_hbm.at[rowbuf.at[j]], gbufs[b],
                                      gsems[b]).wait()
                pltpu.async_copy(gbufs[b], acc.at[dstbuf.at[j]], ssems[b],
                                 add=True)

                bp = (b - 1) % _NBUF
                jp = j - 1

                def _refill():
                    pltpu.make_async_copy(gbufs[bp], acc.at[dstbuf.at[jp]],
                                          ssems[bp]).wait()
                    pltpu.async_copy(table_hbm.at[rowbuf.at[jp + _NBUF]],
                                     gbufs[bp], gsems[bp])

                if b == 0:
                    pl.when((t > 0) & (jp + _NBUF < cpw))(_refill)
                else:
                    pl.when(jp + _NBUF < cpw)(_refill)
            return carry

        lax.fori_loop(0, cpw // _NBUF, _step, 0)

        for b in range(_NBUF):
            j = cpw - _NBUF + b
            pltpu.make_async_copy(gbufs[b], acc.at[dstbuf.at[j]],
                                  ssems[b]).wait()
        plsc.subcore_barrier()

        pltpu.sync_copy(acc.at[pl.ds(s * RPS, RPS)],
                        out_hbm.at[c, pl.ds(s * RPS, RPS)])

    return sc_agg


_BN = 1000


@functools.cache
def _make_pre(di: int):
    def body(h_ref, w_ref, b_ref, o_ref):
        r = pl.program_id(1)
        o_ref[0] = (jnp.dot(h_ref[...], w_ref[0],
                            preferred_element_type=jnp.float32)
                    + b_ref[r][None, :])

    return pl.pallas_call(
        body,
        grid=(N_NODES // _BN, N_ETYPES),
        in_specs=[
            pl.BlockSpec((_BN, di), lambda n, r: (n, 0)),
            pl.BlockSpec((1, di, N_HIDDEN), lambda n, r: (r, 0, 0)),
            pl.BlockSpec((N_ETYPES, N_HIDDEN), lambda n, r: (0, 0)),
        ],
        out_specs=pl.BlockSpec((1, _BN, N_HIDDEN), lambda n, r: (r, n, 0)),
        out_shape=jax.ShapeDtypeStruct((N_ETYPES, N_NODES, N_HIDDEN),
                                       jnp.float32),
    )


@functools.cache
def _make_post(di: int):
    def body(h_ref, p_ref, wg_ref, ug_ref, ws_ref, bg_ref, o_ref):
        hb = h_ref[...]
        agg = p_ref[0] + p_ref[1]
        z = (jnp.dot(hb, wg_ref[...], preferred_element_type=jnp.float32)
             + jnp.dot(agg, ug_ref[...], preferred_element_type=jnp.float32)
             + bg_ref[...][None, :])
        gate = jax.nn.sigmoid(z)
        self_t = jnp.dot(hb, ws_ref[...], preferred_element_type=jnp.float32)
        o_ref[...] = jnp.maximum(gate * agg + (1.0 - gate) * self_t, 0.0)

    return pl.pallas_call(
        body,
        grid=(N_NODES // _BN,),
        in_specs=[
            pl.BlockSpec((_BN, di), lambda n: (n, 0)),
            pl.BlockSpec((NC, _BN, N_HIDDEN), lambda n: (0, n, 0)),
            pl.BlockSpec((di, N_HIDDEN), lambda n: (0, 0)),
            pl.BlockSpec((N_HIDDEN, N_HIDDEN), lambda n: (0, 0)),
            pl.BlockSpec((di, N_HIDDEN), lambda n: (0, 0)),
            pl.BlockSpec((N_HIDDEN,), lambda n: (0,)),
        ],
        out_specs=pl.BlockSpec((_BN, N_HIDDEN), lambda n: (n, 0)),
        out_shape=jax.ShapeDtypeStruct((N_NODES, N_HIDDEN), jnp.float32),
    )


_KB = 25624
_KSTEPS = 25


def _fc1_body(hc_ref, w_ref, o_ref):
    @pl.when(pl.program_id(0) == 0)
    def _():
        o_ref[...] = jnp.zeros_like(o_ref)

    o_ref[...] += lax.dot_general(
        hc_ref[...], w_ref[...], (((0,), (0,)), ((), ())),
        preferred_element_type=jnp.float32)


def _make_fc1(k_total: int):
    assert k_total == _KB * _KSTEPS
    return pl.pallas_call(
        _fc1_body,
        grid=(_KSTEPS,),
        in_specs=[
            pl.BlockSpec((_KB, 1), lambda k: (k, 0)),
            pl.BlockSpec((_KB, N_HIDDEN), lambda k: (k, 0)),
        ],
        out_specs=pl.BlockSpec((1, N_HIDDEN), lambda k: (0, 0)),
        out_shape=jax.ShapeDtypeStruct((1, N_HIDDEN), jnp.float32),
    )


def _head_body(f_ref, b1_ref, w2_ref, b2_ref, w3_ref, b3_ref, o_ref):
    h1 = jnp.maximum(f_ref[...] + b1_ref[...][None, :], 0.0)
    h2 = jnp.maximum(
        jnp.dot(h1, w2_ref[...], preferred_element_type=jnp.float32)
        + b2_ref[...][None, :], 0.0)
    o_ref[...] = jax.nn.sigmoid(
        jnp.dot(h2, w3_ref[...], preferred_element_type=jnp.float32)
        + b3_ref[...][None, :])


def _make_head(n_classes: int):
    return pl.pallas_call(
        _head_body,
        out_shape=jax.ShapeDtypeStruct((1, n_classes), jnp.float32),
    )


def kernel(x, edge_index, edge_type, goalVec, goalObjectsVec, params):
    src = edge_index[0].astype(jnp.int32)
    dst = edge_index[1].astype(jnp.int32)
    et = edge_type.astype(jnp.int32)
    e = src.shape[0]
    cpw = -(-e // (NW * CHUNK))
    cpw = -(-cpw // _NBUF) * _NBUF
    e_pad = NW * cpw * CHUNK

    rows = et * N_NODES + src
    rows = jnp.concatenate(
        [rows, jnp.zeros((e_pad - e,), jnp.int32)]).reshape(NW, cpw, CHUNK)
    dstp = jnp.concatenate(
        [dst, jnp.full((e_pad - e,), DUMMY_DST, jnp.int32)]
    ).reshape(NW, cpw, CHUNK)

    sc_agg = _make_sc_agg(cpw, N_ETYPES * N_NODES)

    h = x
    for p in params['layers']:
        di = h.shape[1]
        table = _make_pre(di)(h, p['W_rel'], p['b_rel'])
        table = table.reshape(N_ETYPES * N_NODES, N_HIDDEN)
        partials = sc_agg(rows, dstp, table)
        h = _make_post(di)(h, partials, p['W_gate'], p['U_gate'],
                           p['W_self'], p['b_gate'])

    hcat = jnp.concatenate([h.reshape(-1), goalVec, goalObjectsVec])
    f1 = _make_fc1(hcat.shape[0])(hcat.reshape(-1, 1), params['fc1_w'])
    out = _make_head(params['fc3_w'].shape[1])(
        f1, params['fc1_b'], params['fc2_w'], params['fc2_b'],
        params['fc3_w'], params['fc3_b'])
    return out.reshape(-1)


# speedup vs baseline: 1.0729x; 1.0729x over previous
import functools

import jax
import jax.numpy as jnp
from jax import lax
from jax.experimental import pallas as pl
from jax.experimental.pallas import tpu as pltpu
from jax.experimental.pallas import tpu_sc as plsc

N_NODES = 10000
N_HIDDEN = 64
N_ETYPES = 4

NC = 2
NS = 16
NW = NC * NS
CHUNK = 128
ACC_ROWS = 10112
RPS = ACC_ROWS // NS
DUMMY_DST = N_NODES


THALF = N_ETYPES * N_NODES // NC
CH = 64
BCH = 4
BLK = CH * BCH
_UP = 1248


@functools.cache
def _make_sc_agg(nb: int):
    assert nb % 2 == 0 and nb >= 2
    mesh = plsc.VectorSubcoreMesh(core_axis_name="c", subcore_axis_name="s")

    @functools.partial(
        pl.kernel,
        mesh=mesh,
        compiler_params=pltpu.CompilerParams(use_tc_tiling_on_sc=False),
        out_type=jax.ShapeDtypeStruct((NC, ACC_ROWS, N_HIDDEN), jnp.float32),
        scratch_types=[
            [pltpu.VMEM((BLK,), jnp.int32)] * 2,
            [pltpu.VMEM((BLK,), jnp.int32)] * 2,
            [pltpu.VMEM((CH,), jnp.int32)] * 2,
            [pltpu.VMEM((CH,), jnp.int32)] * 2,
            [pltpu.VMEM((CH, N_HIDDEN), jnp.float32)] * 2,
            pltpu.VMEM_SHARED((THALF, N_HIDDEN), jnp.float32),
            pltpu.VMEM_SHARED((ACC_ROWS, N_HIDDEN), jnp.float32),
            pltpu.SemaphoreType.DMA,
            [pltpu.SemaphoreType.DMA] * 2,
            [pltpu.SemaphoreType.DMA] * 2,
            [pltpu.SemaphoreType.DMA] * 2,
        ],
    )
    def sc_agg(rows_hbm, dst_hbm, table_hbm, out_hbm,
               rbufs, dbufs, gidx, sdst, gbufs, tbl, acc,
               usem, isems, gsems, ssems):
        c = lax.axis_index("c")
        s = lax.axis_index("s")
        lo = c * THALF
        z16 = jnp.zeros((16,), jnp.float32)

        up_src = table_hbm.at[pl.ds(c * THALF + s * _UP, _UP)]
        up_dst = tbl.at[pl.ds(s * _UP, _UP)]
        pltpu.async_copy(up_src, up_dst, usem)
        rem = THALF - NS * _UP
        rem_src = table_hbm.at[pl.ds(c * THALF + NS * _UP, rem)]
        rem_dst = tbl.at[pl.ds(NS * _UP, rem)]

        @pl.when(s == 0)
        def _():
            pltpu.async_copy(rem_src, rem_dst, usem)

        pltpu.sync_copy(rows_hbm.at[s, 0], rbufs[0])
        pltpu.sync_copy(dst_hbm.at[s, 0], dbufs[0])

        def _zb(i, carry):
            for q in range(N_HIDDEN // 16):
                gbufs[0][i, pl.ds(q * 16, 16)] = z16
            return carry
        lax.fori_loop(0, CH, _zb, 0)
        base = s * RPS
        nfull = RPS // CH
        for t in range(nfull):
            pltpu.sync_copy(gbufs[0], acc.at[pl.ds(base + t * CH, CH)])
        remz = RPS - nfull * CH
        if remz:
            pltpu.sync_copy(gbufs[0].at[pl.ds(0, remz)],
                            acc.at[pl.ds(base + nfull * CH, remz)])

        def _mask(p, rbuf, dbuf, off):
            for v in range(CH // 16):
                rv = rbuf[pl.ds(off + v * 16, 16)] - lo
                ok = rv.astype(jnp.uint32) < jnp.uint32(THALF)
                gidx[p][pl.ds(v * 16, 16)] = jnp.where(ok, rv, 0)
                dv = dbuf[pl.ds(off + v * 16, 16)]
                sdst[p][pl.ds(v * 16, 16)] = jnp.where(ok, dv, DUMMY_DST)

        def _gfire(p):
            pltpu.async_copy(tbl.at[gidx[p]], gbufs[p], gsems[p])

        def _gwait(p):
            pltpu.make_async_copy(tbl.at[gidx[p]], gbufs[p], gsems[p]).wait()

        def _sfire(p):
            pltpu.async_copy(gbufs[p], acc.at[sdst[p]], ssems[p], add=True)

        def _swait(p):
            pltpu.make_async_copy(gbufs[p], acc.at[sdst[p]], ssems[p]).wait()

        def _fire_load(b, q):
            pltpu.async_copy(rows_hbm.at[s, b], rbufs[q], isems[q])
            pltpu.async_copy(dst_hbm.at[s, b], dbufs[q], isems[q])

        def _wait_load(b, q):
            pltpu.make_async_copy(rows_hbm.at[s, b], rbufs[q],
                                  isems[q]).wait()
            pltpu.make_async_copy(dst_hbm.at[s, b], dbufs[q],
                                  isems[q]).wait()

        pltpu.make_async_copy(up_src, up_dst, usem).wait()

        @pl.when(s == 0)
        def _():
            pltpu.make_async_copy(rem_src, rem_dst, usem).wait()
        plsc.subcore_barrier()

        _mask(0, rbufs[0], dbufs[0], 0)
        _mask(1, rbufs[0], dbufs[0], CH)
        _gfire(0)
        _gfire(1)

        def _deferred(rbuf, dbuf):
            _swait(0)
            _mask(0, rbuf, dbuf, 0)
            _gfire(0)

        def _steps(b, rbuf, dbuf, guard_first):
            for c4 in range(BCH):
                p = c4 % 2
                _gwait(p)
                _sfire(p)
                if c4 < BCH - 1:
                    pp = 1 - p

                    def _rf(pp=pp, off=(c4 + 1) * CH):
                        _swait(pp)
                        _mask(pp, rbuf, dbuf, off)
                        _gfire(pp)

                    if guard_first and c4 == 0:
                        pl.when(b > 0)(_rf)
                    else:
                        _rf()

        def _body(bb, carry):
            bA = 2 * bb

            @pl.when(bb > 0)
            def _():
                _wait_load(bA, 0)
                _deferred(rbufs[0], dbufs[0])

            _fire_load(bA + 1, 1)
            _steps(bA, rbufs[0], dbufs[0], True)

            bB = bA + 1
            _wait_load(bB, 1)
            _deferred(rbufs[1], dbufs[1])

            @pl.when(bb < nb // 2 - 1)
            def _():
                _fire_load(bB + 1, 0)
            _steps(bB, rbufs[1], dbufs[1], False)
            return carry

        lax.fori_loop(0, nb // 2, _body, 0)

        _swait(0)
        _swait(1)
        plsc.subcore_barrier()

        pltpu.sync_copy(acc.at[pl.ds(s * RPS, RPS)],
                        out_hbm.at[c, pl.ds(s * RPS, RPS)])

    return sc_agg


_BN = 1000


@functools.cache
def _make_pre(di: int):
    def body(h_ref, w_ref, b_ref, o_ref):
        r = pl.program_id(1)
        o_ref[0] = (jnp.dot(h_ref[...], w_ref[0],
                            preferred_element_type=jnp.float32)
                    + b_ref[r][None, :])

    return pl.pallas_call(
        body,
        grid=(N_NODES // _BN, N_ETYPES),
        in_specs=[
            pl.BlockSpec((_BN, di), lambda n, r: (n, 0)),
            pl.BlockSpec((1, di, N_HIDDEN), lambda n, r: (r, 0, 0)),
            pl.BlockSpec((N_ETYPES, N_HIDDEN), lambda n, r: (0, 0)),
        ],
        out_specs=pl.BlockSpec((1, _BN, N_HIDDEN), lambda n, r: (r, n, 0)),
        out_shape=jax.ShapeDtypeStruct((N_ETYPES, N_NODES, N_HIDDEN),
                                       jnp.float32),
    )


@functools.cache
def _make_post(di: int):
    def body(h_ref, p_ref, wg_ref, ug_ref, ws_ref, bg_ref, o_ref):
        hb = h_ref[...]
        agg = p_ref[0] + p_ref[1]
        z = (jnp.dot(hb, wg_ref[...], preferred_element_type=jnp.float32)
             + jnp.dot(agg, ug_ref[...], preferred_element_type=jnp.float32)
             + bg_ref[...][None, :])
        gate = jax.nn.sigmoid(z)
        self_t = jnp.dot(hb, ws_ref[...], preferred_element_type=jnp.float32)
        o_ref[...] = jnp.maximum(gate * agg + (1.0 - gate) * self_t, 0.0)

    return pl.pallas_call(
        body,
        grid=(N_NODES // _BN,),
        in_specs=[
            pl.BlockSpec((_BN, di), lambda n: (n, 0)),
            pl.BlockSpec((NC, _BN, N_HIDDEN), lambda n: (0, n, 0)),
            pl.BlockSpec((di, N_HIDDEN), lambda n: (0, 0)),
            pl.BlockSpec((N_HIDDEN, N_HIDDEN), lambda n: (0, 0)),
            pl.BlockSpec((di, N_HIDDEN), lambda n: (0, 0)),
            pl.BlockSpec((N_HIDDEN,), lambda n: (0,)),
        ],
        out_specs=pl.BlockSpec((_BN, N_HIDDEN), lambda n: (n, 0)),
        out_shape=jax.ShapeDtypeStruct((N_NODES, N_HIDDEN), jnp.float32),
    )


_KB = 25624
_KSTEPS = 25


def _fc1_body(hc_ref, w_ref, o_ref):
    @pl.when(pl.program_id(0) == 0)
    def _():
        o_ref[...] = jnp.zeros_like(o_ref)

    o_ref[...] += lax.dot_general(
        hc_ref[...], w_ref[...], (((0,), (0,)), ((), ())),
        preferred_element_type=jnp.float32)


def _make_fc1(k_total: int):
    assert k_total == _KB * _KSTEPS
    return pl.pallas_call(
        _fc1_body,
        grid=(_KSTEPS,),
        in_specs=[
            pl.BlockSpec((_KB, 1), lambda k: (k, 0)),
            pl.BlockSpec((_KB, N_HIDDEN), lambda k: (k, 0)),
        ],
        out_specs=pl.BlockSpec((1, N_HIDDEN), lambda k: (0, 0)),
        out_shape=jax.ShapeDtypeStruct((1, N_HIDDEN), jnp.float32),
    )


def _head_body(f_ref, b1_ref, w2_ref, b2_ref, w3_ref, b3_ref, o_ref):
    h1 = jnp.maximum(f_ref[...] + b1_ref[...][None, :], 0.0)
    h2 = jnp.maximum(
        jnp.dot(h1, w2_ref[...], preferred_element_type=jnp.float32)
        + b2_ref[...][None, :], 0.0)
    o_ref[...] = jax.nn.sigmoid(
        jnp.dot(h2, w3_ref[...], preferred_element_type=jnp.float32)
        + b3_ref[...][None, :])


def _make_head(n_classes: int):
    return pl.pallas_call(
        _head_body,
        out_shape=jax.ShapeDtypeStruct((1, n_classes), jnp.float32),
    )


def kernel(x, edge_index, edge_type, goalVec, goalObjectsVec, params):
    src = edge_index[0].astype(jnp.int32)
    dst = edge_index[1].astype(jnp.int32)
    et = edge_type.astype(jnp.int32)
    e = src.shape[0]
    nb = -(-e // (NS * BLK))
    nb += nb % 2
    e_pad = NS * nb * BLK

    rows = et * N_NODES + src
    rows = jnp.concatenate(
        [rows, jnp.zeros((e_pad - e,), jnp.int32)]).reshape(NS, nb, BLK)
    dstp = jnp.concatenate(
        [dst, jnp.full((e_pad - e,), DUMMY_DST, jnp.int32)]
    ).reshape(NS, nb, BLK)

    sc_agg = _make_sc_agg(nb)

    h = x
    for p in params['layers']:
        di = h.shape[1]
        table = _make_pre(di)(h, p['W_rel'], p['b_rel'])
        table = table.reshape(N_ETYPES * N_NODES, N_HIDDEN)
        partials = sc_agg(rows, dstp, table)
        h = _make_post(di)(h, partials, p['W_gate'], p['U_gate'],
                           p['W_self'], p['b_gate'])

    hcat = jnp.concatenate([h.reshape(-1), goalVec, goalObjectsVec])
    f1 = _make_fc1(hcat.shape[0])(hcat.reshape(-1, 1), params['fc1_w'])
    out = _make_head(params['fc3_w'].shape[1])(
        f1, params['fc1_b'], params['fc2_w'], params['fc2_b'],
        params['fc3_w'], params['fc3_b'])
    return out.reshape(-1)
